# baseline (device time: 54001 ns/iter reference)
import jax
import jax.numpy as jnp
from jax import lax
from jax.experimental import pallas as pl
from jax.experimental.pallas import tpu as pltpu

N_DEV = 4


def kernel(x):
    m, n = x.shape

    def body(x_ref, out_ref, gather_ref, send_sems, recv_sems, ack_sem):
        my = lax.axis_index("i")

        xv = x_ref[...]
        tot = xv
        h = m
        while h > 1:
            h //= 2
            tot = tot[:h, :] * tot[h:, :]
        gather_ref[0, :] = tot[0, :]

        copies = []
        for k in range(1, N_DEV):
            rdma = pltpu.make_async_remote_copy(
                src_ref=gather_ref.at[0],
                dst_ref=gather_ref.at[k],
                send_sem=send_sems.at[k - 1],
                recv_sem=recv_sems.at[k - 1],
                device_id=((my + k) % N_DEV,),
                device_id_type=pl.DeviceIdType.MESH,
            )
            rdma.start()
            copies.append(rdma)

        acc = xv
        s = 1
        while s < m:
            shifted = jnp.concatenate(
                [jnp.ones((s, n), jnp.float32), acc[: m - s, :]], axis=0
            )
            acc = acc * shifted
            s *= 2

        for rdma in copies:
            rdma.wait_send()
            rdma.wait_recv()

        g = gather_ref[...]
        ones = jnp.ones((n,), jnp.float32)
        pfx = ones
        for k in range(1, N_DEV):
            pfx = pfx * jnp.where(my >= k, g[k], ones)

        out_ref[...] = acc * pfx

        for k in range(1, N_DEV):
            pl.semaphore_signal(
                ack_sem,
                inc=1,
                device_id=((my + k) % N_DEV,),
                device_id_type=pl.DeviceIdType.MESH,
            )
        pl.semaphore_wait(ack_sem, N_DEV - 1)

    return pl.pallas_call(
        body,
        out_shape=jax.ShapeDtypeStruct((m, n), jnp.float32),
        in_specs=[pl.BlockSpec(memory_space=pltpu.VMEM)],
        out_specs=pl.BlockSpec(memory_space=pltpu.VMEM),
        scratch_shapes=[
            pltpu.VMEM((N_DEV, n), jnp.float32),
            pltpu.SemaphoreType.DMA((N_DEV - 1,)),
            pltpu.SemaphoreType.DMA((N_DEV - 1,)),
            pltpu.SemaphoreType.REGULAR,
        ],
        compiler_params=pltpu.CompilerParams(
            vmem_limit_bytes=100 * 1024 * 1024,
        ),
    )(x)
